# Initial kernel scaffold; baseline (speedup 1.0000x reference)
import jax
import jax.numpy as jnp
from jax.experimental import pallas as pl


def kernel(xyz, features, sample_inds):
    raise NotImplementedError("write your pallas kernel here")


# speedup vs baseline: 1.0228x; 1.0228x over previous
import functools

import jax
import jax.numpy as jnp
from jax import lax
from jax.experimental import pallas as pl
from jax.experimental.pallas import tpu as pltpu
from jax.experimental.pallas import tpu_sc as plsc

_L = 16


def _build_sc_gather(B, N, C, M):
    info = plsc.get_sparse_core_info()
    NC, NS = info.num_cores, info.num_subcores
    NW = NC * NS
    assert NW == 2 * B, "mapping assumes 2 tiles per batch"
    HC = C // 2
    HM = M // 2

    mesh = plsc.VectorSubcoreMesh(core_axis_name="c", subcore_axis_name="s")

    @functools.partial(
        pl.kernel,
        mesh=mesh,
        compiler_params=pltpu.CompilerParams(needs_layout_passes=False),
        out_type=(
            jax.ShapeDtypeStruct((B * M * 3,), jnp.float32),
            jax.ShapeDtypeStruct((B * C * M,), jnp.float32),
        ),
        scratch_types=[
            pltpu.VMEM((M,), jnp.int32),
            pltpu.VMEM((N * 3,), jnp.float32),
            pltpu.VMEM((HM * 3,), jnp.float32),
            pltpu.VMEM((N,), jnp.float32),
            pltpu.VMEM((M,), jnp.float32),
        ],
    )
    def sc_gather(xyz_hbm, feat_hbm, inds_hbm, oxyz_hbm, ofeat_hbm,
                  idx_v, xyzb_v, xyzout_v, row_v, fout_v):
        wid = lax.axis_index("s") * NC + lax.axis_index("c")
        b = wid // 2
        h = wid % 2

        pltpu.sync_copy(inds_hbm.at[pl.ds(b * M, M)], idx_v)

        pltpu.sync_copy(xyz_hbm.at[pl.ds(b * N * 3, N * 3)], xyzb_v)
        iota = lax.iota(jnp.int32, _L)

        def xyz_block(j, carry):
            inds16 = idx_v[pl.ds(h * HM + j * _L, _L)]
            src = inds16 * 3
            dst = (j * _L + iota) * 3
            for d in range(3):
                vals = plsc.load_gather(xyzb_v, [src + d])
                plsc.store_scatter(xyzout_v, [dst + d], vals)
            return carry

        lax.fori_loop(0, HM // _L, xyz_block, 0)
        pltpu.sync_copy(xyzout_v,
                        oxyz_hbm.at[pl.ds((b * M + h * HM) * 3, HM * 3)])

        def chan(c, carry):
            r = b * C + h * HC + c
            pltpu.sync_copy(feat_hbm.at[pl.ds(r * N, N)], row_v)

            def mblock(j, carry2):
                idx16 = idx_v[pl.ds(j * _L, _L)]
                fout_v[pl.ds(j * _L, _L)] = plsc.load_gather(row_v, [idx16])
                return carry2

            lax.fori_loop(0, M // _L, mblock, 0)
            pltpu.sync_copy(fout_v, ofeat_hbm.at[pl.ds(r * M, M)])
            return carry

        lax.fori_loop(0, HC, chan, 0)

    return sc_gather


def kernel(xyz, features, sample_inds):
    B, N, _ = xyz.shape
    _, C, _ = features.shape
    M = sample_inds.shape[1]
    sc_gather = _build_sc_gather(B, N, C, M)
    oxyz, ofeat = sc_gather(
        xyz.reshape(-1),
        features.reshape(-1),
        sample_inds.astype(jnp.int32).reshape(-1),
    )
    return (oxyz.reshape(B, M, 3), ofeat.reshape(B, C, M), sample_inds)


# speedup vs baseline: 1.5109x; 1.4772x over previous
import functools

import jax
import jax.numpy as jnp
from jax import lax
from jax.experimental import pallas as pl
from jax.experimental.pallas import tpu as pltpu
from jax.experimental.pallas import tpu_sc as plsc

_L = 16


def _build_sc_gather(B, N, C, M):
    info = plsc.get_sparse_core_info()
    NC, NS = info.num_cores, info.num_subcores
    NW = NC * NS
    assert NW == 2 * B, "mapping assumes 2 tiles per batch"
    HC = C // 2
    HM = M // 2

    mesh = plsc.VectorSubcoreMesh(core_axis_name="c", subcore_axis_name="s")

    @functools.partial(
        pl.kernel,
        mesh=mesh,
        compiler_params=pltpu.CompilerParams(needs_layout_passes=False),
        out_type=(
            jax.ShapeDtypeStruct((B * M * 3,), jnp.float32),
            jax.ShapeDtypeStruct((B * C * M,), jnp.float32),
        ),
        scratch_types=[
            pltpu.VMEM((M,), jnp.int32),
            pltpu.VMEM((N * 3,), jnp.float32),
            pltpu.VMEM((HM * 3,), jnp.float32),
            pltpu.VMEM((N,), jnp.float32),
            pltpu.VMEM((N,), jnp.float32),
            pltpu.VMEM((M,), jnp.float32),
            pltpu.VMEM((M,), jnp.float32),
            pltpu.SemaphoreType.DMA,
            pltpu.SemaphoreType.DMA,
            pltpu.SemaphoreType.DMA,
            pltpu.SemaphoreType.DMA,
        ],
    )
    def sc_gather(xyz_hbm, feat_hbm, inds_hbm, oxyz_hbm, ofeat_hbm,
                  idx_v, xyzb_v, xyzout_v, row0_v, row1_v, fout0_v, fout1_v,
                  sin0, sin1, sout0, sout1):
        wid = lax.axis_index("s") * NC + lax.axis_index("c")
        b = wid // 2
        h = wid % 2
        rows = (row0_v, row1_v)
        fouts = (fout0_v, fout1_v)
        in_sems = (sin0, sin1)
        out_sems = (sout0, sout1)

        pltpu.sync_copy(inds_hbm.at[pl.ds(b * M, M)], idx_v)

        def row_src(c):
            return feat_hbm.at[pl.ds((b * C + h * HC + c) * N, N)]

        def out_dst(c):
            return ofeat_hbm.at[pl.ds((b * C + h * HC + c) * M, M)]

        for k in range(2):
            pltpu.make_async_copy(row_src(k), rows[k], in_sems[k]).start()

        pltpu.sync_copy(xyz_hbm.at[pl.ds(b * N * 3, N * 3)], xyzb_v)
        iota = lax.iota(jnp.int32, _L)

        @plsc.parallel_loop(0, HM // _L, unroll=4)
        def xyz_block(j):
            inds16 = idx_v[pl.ds(h * HM + j * _L, _L)]
            src = inds16 * 3
            dst = (j * _L + iota) * 3
            for d in range(3):
                vals = plsc.load_gather(xyzb_v, [src + d])
                plsc.store_scatter(xyzout_v, [dst + d], vals)

        pltpu.sync_copy(xyzout_v,
                        oxyz_hbm.at[pl.ds((b * M + h * HM) * 3, HM * 3)])

        def chan_pair(i, carry):
            for k in range(2):
                c = i * 2 + k
                pltpu.make_async_copy(row_src(0), rows[k],
                                      in_sems[k]).wait()

                @pl.when(i > 0)
                def _wait_out():
                    pltpu.make_async_copy(fouts[k], out_dst(0),
                                          out_sems[k]).wait()

                @plsc.parallel_loop(0, M // _L, unroll=8)
                def gather_block(j):
                    idx16 = idx_v[pl.ds(j * _L, _L)]
                    fouts[k][pl.ds(j * _L, _L)] = plsc.load_gather(
                        rows[k], [idx16])

                pltpu.make_async_copy(fouts[k], out_dst(c),
                                      out_sems[k]).start()

                @pl.when(c + 2 < HC)
                def _next_in():
                    pltpu.make_async_copy(row_src(c + 2), rows[k],
                                          in_sems[k]).start()
            return carry

        lax.fori_loop(0, HC // 2, chan_pair, 0)
        for k in range(2):
            pltpu.make_async_copy(fouts[k], out_dst(0), out_sems[k]).wait()

    return sc_gather


def kernel(xyz, features, sample_inds):
    B, N, _ = xyz.shape
    _, C, _ = features.shape
    M = sample_inds.shape[1]
    sc_gather = _build_sc_gather(B, N, C, M)
    oxyz, ofeat = sc_gather(
        xyz.reshape(-1),
        features.reshape(-1),
        sample_inds.astype(jnp.int32).reshape(-1),
    )
    return (oxyz.reshape(B, M, 3), ofeat.reshape(B, C, M), sample_inds)


# speedup vs baseline: 5.2597x; 3.4813x over previous
import functools

import jax
import jax.numpy as jnp
from jax import lax
from jax.experimental import pallas as pl
from jax.experimental.pallas import tpu as pltpu
from jax.experimental.pallas import tpu_sc as plsc

_L = 16


def _build_sc_gather(B, N, C, M):
    info = plsc.get_sparse_core_info()
    NC, NS = info.num_cores, info.num_subcores
    NW = NC * NS
    assert NW == 2 * B, "mapping assumes 2 tiles per batch"
    HC = C // 2
    HM = M // 2
    NT = N // 128
    MT = M // 128
    R = B * C // 8

    mesh = plsc.VectorSubcoreMesh(core_axis_name="c", subcore_axis_name="s")

    @functools.partial(
        pl.kernel,
        mesh=mesh,
        compiler_params=pltpu.CompilerParams(needs_layout_passes=False),
        out_type=(
            jax.ShapeDtypeStruct((3, B // 8, MT, 8, 128), jnp.float32),
            jax.ShapeDtypeStruct((R, MT, 8, 128), jnp.float32),
        ),
        scratch_types=[
            pltpu.VMEM((MT, 128), jnp.int32),
            pltpu.VMEM((NT, 128), jnp.float32),
            pltpu.VMEM((NT, 128), jnp.float32),
            pltpu.VMEM((NT, 128), jnp.float32),
            pltpu.VMEM((MT // 2, 128), jnp.float32),
            pltpu.VMEM((MT // 2, 128), jnp.float32),
            pltpu.VMEM((MT // 2, 128), jnp.float32),
            pltpu.VMEM((NT, 128), jnp.float32),
            pltpu.VMEM((NT, 128), jnp.float32),
            pltpu.VMEM((MT, 128), jnp.float32),
            pltpu.VMEM((MT, 128), jnp.float32),
            pltpu.SemaphoreType.DMA,
            pltpu.SemaphoreType.DMA,
            pltpu.SemaphoreType.DMA,
            pltpu.SemaphoreType.DMA,
        ],
    )
    def sc_gather(xyz_hbm, feat_hbm, inds_hbm, oxyz_hbm, ofeat_hbm,
                  idx_v, xp0_v, xp1_v, xp2_v, xo0_v, xo1_v, xo2_v,
                  row0_v, row1_v, fout0_v, fout1_v,
                  sin0, sin1, sout0, sout1):
        wid = lax.axis_index("s") * NC + lax.axis_index("c")
        b = wid // 2
        h = wid % 2
        bt, bs = b // 8, b % 8
        planes = (xp0_v, xp1_v, xp2_v)
        xouts = (xo0_v, xo1_v, xo2_v)
        rows = (row0_v, row1_v)
        fouts = (fout0_v, fout1_v)
        in_sems = (sin0, sin1)
        out_sems = (sout0, sout1)
        r_base = b * (C // 8) + h * (HC // 8)

        pltpu.sync_copy(inds_hbm.at[bt, :, bs, :], idx_v)

        def row_src(c):
            return feat_hbm.at[r_base + c // 8, :, c % 8, :]

        def out_dst(c):
            return ofeat_hbm.at[r_base + c // 8, :, c % 8, :]

        for k in range(2):
            pltpu.make_async_copy(row_src(k), rows[k], in_sems[k]).start()

        for d in range(3):
            pltpu.sync_copy(xyz_hbm.at[d, bt, :, bs, :], planes[d])

        @plsc.parallel_loop(0, HM // _L, unroll=4)
        def xyz_block(j):
            idx16 = idx_v[h * (MT // 2) + j // 8, pl.ds((j % 8) * _L, _L)]
            hi = idx16 >> 7
            lo = idx16 & 127
            for d in range(3):
                vals = plsc.load_gather(planes[d], [hi, lo])
                xouts[d][j // 8, pl.ds((j % 8) * _L, _L)] = vals

        for d in range(3):
            pltpu.sync_copy(xouts[d],
                            oxyz_hbm.at[d, bt, pl.ds(h * (MT // 2), MT // 2),
                                        bs, :])

        def chan_pair(i, carry):
            for k in range(2):
                c = i * 2 + k
                pltpu.make_async_copy(row_src(0), rows[k],
                                      in_sems[k]).wait()

                @pl.when(i > 0)
                def _wait_out():
                    pltpu.make_async_copy(fouts[k], out_dst(0),
                                          out_sems[k]).wait()

                @plsc.parallel_loop(0, M // _L, unroll=8)
                def gather_block(j):
                    idx16 = idx_v[j // 8, pl.ds((j % 8) * _L, _L)]
                    vals = plsc.load_gather(rows[k], [idx16 >> 7, idx16 & 127])
                    fouts[k][j // 8, pl.ds((j % 8) * _L, _L)] = vals

                pltpu.make_async_copy(fouts[k], out_dst(c),
                                      out_sems[k]).start()

                @pl.when(c + 2 < HC)
                def _next_in():
                    pltpu.make_async_copy(row_src(c + 2), rows[k],
                                          in_sems[k]).start()
            return carry

        lax.fori_loop(0, HC // 2, chan_pair, 0)
        for k in range(2):
            pltpu.make_async_copy(fouts[k], out_dst(0), out_sems[k]).wait()

    return sc_gather


def kernel(xyz, features, sample_inds):
    B, N, _ = xyz.shape
    _, C, _ = features.shape
    M = sample_inds.shape[1]
    sc_gather = _build_sc_gather(B, N, C, M)
    feat4 = features.reshape(B * C // 8, 8, N // 128, 128).transpose(0, 2, 1, 3)
    inds4 = (sample_inds.astype(jnp.int32)
             .reshape(B // 8, 8, M // 128, 128).transpose(0, 2, 1, 3))
    xyz5 = (xyz.transpose(2, 0, 1)
            .reshape(3, B // 8, 8, N // 128, 128).transpose(0, 1, 3, 2, 4))
    oxyz5, ofeat4 = sc_gather(xyz5, feat4, inds4)
    new_xyz = (oxyz5.transpose(0, 1, 3, 2, 4)
               .reshape(3, B, M).transpose(1, 2, 0))
    new_features = ofeat4.transpose(0, 2, 1, 3).reshape(B, C, M)
    return (new_xyz, new_features, sample_inds)


# speedup vs baseline: 6.2276x; 1.1840x over previous
import functools

import jax
import jax.numpy as jnp
from jax import lax
from jax.experimental import pallas as pl
from jax.experimental.pallas import tpu as pltpu
from jax.experimental.pallas import tpu_sc as plsc

_L = 16
_NB = 4


def _build_sc_gather(B, N, C, M):
    info = plsc.get_sparse_core_info()
    NC, NS = info.num_cores, info.num_subcores
    NW = NC * NS
    assert NW == 2 * B, "mapping assumes 2 tiles per batch"
    HC = C // 2
    HM = M // 2
    NT = N // 128
    MT = M // 128
    R = B * C // 8

    mesh = plsc.VectorSubcoreMesh(core_axis_name="c", subcore_axis_name="s")

    @functools.partial(
        pl.kernel,
        mesh=mesh,
        compiler_params=pltpu.CompilerParams(needs_layout_passes=False),
        out_type=(
            jax.ShapeDtypeStruct((3, B // 8, MT, 8, 128), jnp.float32),
            jax.ShapeDtypeStruct((R, MT, 8, 128), jnp.float32),
        ),
        scratch_types=[
            pltpu.VMEM((MT, 128), jnp.int32)]
        + [pltpu.VMEM((MT // 2, 128), jnp.float32)
           for _ in range(3)]
        + [pltpu.VMEM((NT, 128), jnp.float32)
           for _ in range(_NB)]
        + [pltpu.VMEM((MT, 128), jnp.float32)
           for _ in range(_NB)]
        + [pltpu.SemaphoreType.DMA for _ in range(2 * _NB)],
    )
    def sc_gather(xyz_hbm, feat_hbm, inds_hbm, oxyz_hbm, ofeat_hbm, *refs):
        idx_v = refs[0]
        xouts = refs[1:4]
        rows = refs[4:4 + _NB]
        fouts = refs[4 + _NB:4 + 2 * _NB]
        in_sems = refs[4 + 2 * _NB:4 + 3 * _NB]
        out_sems = refs[4 + 3 * _NB:4 + 4 * _NB]

        wid = lax.axis_index("s") * NC + lax.axis_index("c")
        b = wid // 2
        h = wid % 2
        bt, bs = b // 8, b % 8
        r_base = b * (C // 8) + h * (HC // 8)

        pltpu.sync_copy(inds_hbm.at[bt, :, bs, :], idx_v)

        def row_src(c):
            return feat_hbm.at[r_base + c // 8, :, c % 8, :]

        def out_dst(c):
            return ofeat_hbm.at[r_base + c // 8, :, c % 8, :]

        for d in range(3):
            pltpu.make_async_copy(xyz_hbm.at[d, bt, :, bs, :], rows[d],
                                  in_sems[d]).start()
        for d in range(3):
            pltpu.make_async_copy(xyz_hbm.at[d, bt, :, bs, :], rows[d],
                                  in_sems[d]).wait()

        @plsc.parallel_loop(0, HM // _L, unroll=4)
        def xyz_block(j):
            idx16 = idx_v[h * (MT // 2) + j // 8, pl.ds((j % 8) * _L, _L)]
            hi = idx16 >> 7
            lo = idx16 & 127
            for d in range(3):
                vals = plsc.load_gather(rows[d], [hi, lo])
                xouts[d][j // 8, pl.ds((j % 8) * _L, _L)] = vals

        for d in range(3):
            pltpu.sync_copy(xouts[d],
                            oxyz_hbm.at[d, bt, pl.ds(h * (MT // 2), MT // 2),
                                        bs, :])

        for k in range(_NB):
            pltpu.make_async_copy(row_src(k), rows[k], in_sems[k]).start()

        def chan_group(i, carry):
            for k in range(_NB):
                c = i * _NB + k
                pltpu.make_async_copy(row_src(0), rows[k],
                                      in_sems[k]).wait()

                @pl.when(i > 0)
                def _wait_out():
                    pltpu.make_async_copy(fouts[k], out_dst(0),
                                          out_sems[k]).wait()

                @plsc.parallel_loop(0, M // _L, unroll=8)
                def gather_block(j):
                    idx16 = idx_v[j // 8, pl.ds((j % 8) * _L, _L)]
                    vals = plsc.load_gather(rows[k], [idx16 >> 7, idx16 & 127])
                    fouts[k][j // 8, pl.ds((j % 8) * _L, _L)] = vals

                pltpu.make_async_copy(fouts[k], out_dst(c),
                                      out_sems[k]).start()

                @pl.when(c + _NB < HC)
                def _next_in():
                    pltpu.make_async_copy(row_src(c + _NB), rows[k],
                                          in_sems[k]).start()
            return carry

        lax.fori_loop(0, HC // _NB, chan_group, 0)
        for k in range(_NB):
            pltpu.make_async_copy(fouts[k], out_dst(0), out_sems[k]).wait()

    return sc_gather


def kernel(xyz, features, sample_inds):
    B, N, _ = xyz.shape
    _, C, _ = features.shape
    M = sample_inds.shape[1]
    sc_gather = _build_sc_gather(B, N, C, M)
    feat4 = features.reshape(B * C // 8, 8, N // 128, 128).transpose(0, 2, 1, 3)
    inds4 = (sample_inds.astype(jnp.int32)
             .reshape(B // 8, 8, M // 128, 128).transpose(0, 2, 1, 3))
    xyz5 = (xyz.transpose(2, 0, 1)
            .reshape(3, B // 8, 8, N // 128, 128).transpose(0, 1, 3, 2, 4))
    oxyz5, ofeat4 = sc_gather(xyz5, feat4, inds4)
    new_xyz = (oxyz5.transpose(0, 1, 3, 2, 4)
               .reshape(3, B, M).transpose(1, 2, 0))
    new_features = ofeat4.transpose(0, 2, 1, 3).reshape(B, C, M)
    return (new_xyz, new_features, sample_inds)


# speedup vs baseline: 6.2642x; 1.0059x over previous
import functools

import jax
import jax.numpy as jnp
from jax import lax
from jax.experimental import pallas as pl
from jax.experimental.pallas import tpu as pltpu
from jax.experimental.pallas import tpu_sc as plsc

_L = 16
_NB = 4


def _build_sc_gather(B, N, C, M):
    info = plsc.get_sparse_core_info()
    NC, NS = info.num_cores, info.num_subcores
    NW = NC * NS
    assert NW == 2 * B, "mapping assumes 2 tiles per batch"
    HC = C // 2
    HM = M // 2
    NT = N // 128
    MT = M // 128
    R = B * C // 8

    mesh = plsc.VectorSubcoreMesh(core_axis_name="c", subcore_axis_name="s")

    @functools.partial(
        pl.kernel,
        mesh=mesh,
        compiler_params=pltpu.CompilerParams(needs_layout_passes=False),
        out_type=(
            jax.ShapeDtypeStruct((3, B // 8, MT, 8, 128), jnp.float32),
            jax.ShapeDtypeStruct((R, MT, 8, 128), jnp.float32),
            jax.ShapeDtypeStruct((B // 8, MT, 8, 128), jnp.int32),
        ),
        scratch_types=[
            pltpu.VMEM((MT, 128), jnp.int32)]
        + [pltpu.VMEM((MT // 2, 128), jnp.float32)
           for _ in range(3)]
        + [pltpu.VMEM((NT, 128), jnp.float32)
           for _ in range(_NB)]
        + [pltpu.VMEM((MT, 128), jnp.float32)
           for _ in range(_NB)]
        + [pltpu.SemaphoreType.DMA for _ in range(2 * _NB)],
    )
    def sc_gather(xyz_hbm, feat_hbm, inds_hbm, oxyz_hbm, ofeat_hbm,
                  oinds_hbm, *refs):
        idx_v = refs[0]
        xouts = refs[1:4]
        rows = refs[4:4 + _NB]
        fouts = refs[4 + _NB:4 + 2 * _NB]
        in_sems = refs[4 + 2 * _NB:4 + 3 * _NB]
        out_sems = refs[4 + 3 * _NB:4 + 4 * _NB]

        wid = lax.axis_index("s") * NC + lax.axis_index("c")
        b = wid // 2
        h = wid % 2
        bt, bs = b // 8, b % 8
        r_base = b * (C // 8) + h * (HC // 8)

        pltpu.sync_copy(inds_hbm.at[bt, :, bs, :], idx_v)

        @pl.when(h == 0)
        def _inds_out():
            pltpu.sync_copy(idx_v, oinds_hbm.at[bt, :, bs, :])

        def row_src(c):
            return feat_hbm.at[r_base + c // 8, :, c % 8, :]

        def out_dst(c):
            return ofeat_hbm.at[r_base + c // 8, :, c % 8, :]

        for d in range(3):
            pltpu.make_async_copy(xyz_hbm.at[d, bt, :, bs, :], rows[d],
                                  in_sems[d]).start()
        for d in range(3):
            pltpu.make_async_copy(xyz_hbm.at[d, bt, :, bs, :], rows[d],
                                  in_sems[d]).wait()

        @plsc.parallel_loop(0, HM // _L, unroll=4)
        def xyz_block(j):
            idx16 = idx_v[h * (MT // 2) + j // 8, pl.ds((j % 8) * _L, _L)]
            hi = idx16 >> 7
            lo = idx16 & 127
            for d in range(3):
                vals = plsc.load_gather(rows[d], [hi, lo])
                xouts[d][j // 8, pl.ds((j % 8) * _L, _L)] = vals

        for d in range(3):
            pltpu.sync_copy(xouts[d],
                            oxyz_hbm.at[d, bt, pl.ds(h * (MT // 2), MT // 2),
                                        bs, :])

        for k in range(_NB):
            pltpu.make_async_copy(row_src(k), rows[k], in_sems[k]).start()

        def chan_group(i, carry):
            for k in range(_NB):
                c = i * _NB + k
                pltpu.make_async_copy(row_src(0), rows[k],
                                      in_sems[k]).wait()

                @pl.when(i > 0)
                def _wait_out():
                    pltpu.make_async_copy(fouts[k], out_dst(0),
                                          out_sems[k]).wait()

                @plsc.parallel_loop(0, M // _L, unroll=8)
                def gather_block(j):
                    idx16 = idx_v[j // 8, pl.ds((j % 8) * _L, _L)]
                    vals = plsc.load_gather(rows[k], [idx16 >> 7, idx16 & 127])
                    fouts[k][j // 8, pl.ds((j % 8) * _L, _L)] = vals

                pltpu.make_async_copy(fouts[k], out_dst(c),
                                      out_sems[k]).start()

                @pl.when(c + _NB < HC)
                def _next_in():
                    pltpu.make_async_copy(row_src(c + _NB), rows[k],
                                          in_sems[k]).start()
            return carry

        lax.fori_loop(0, HC // _NB, chan_group, 0)
        for k in range(_NB):
            pltpu.make_async_copy(fouts[k], out_dst(0), out_sems[k]).wait()

    return sc_gather


def kernel(xyz, features, sample_inds):
    B, N, _ = xyz.shape
    _, C, _ = features.shape
    M = sample_inds.shape[1]
    sc_gather = _build_sc_gather(B, N, C, M)
    feat4 = features.reshape(B * C // 8, 8, N // 128, 128).transpose(0, 2, 1, 3)
    inds4 = (sample_inds.astype(jnp.int32)
             .reshape(B // 8, 8, M // 128, 128).transpose(0, 2, 1, 3))
    xyz5 = (xyz.transpose(2, 0, 1)
            .reshape(3, B // 8, 8, N // 128, 128).transpose(0, 1, 3, 2, 4))
    oxyz5, ofeat4, oinds4 = sc_gather(xyz5, feat4, inds4)
    new_xyz = (oxyz5.transpose(0, 1, 3, 2, 4)
               .reshape(3, B, M).transpose(1, 2, 0))
    new_features = ofeat4.transpose(0, 2, 1, 3).reshape(B, C, M)
    out_inds = (oinds4.transpose(0, 2, 1, 3).reshape(B, M)
                .astype(sample_inds.dtype))
    return (new_xyz, new_features, out_inds)
